# Initial kernel scaffold; baseline (speedup 1.0000x reference)
import jax
import jax.numpy as jnp
from jax.experimental import pallas as pl


def kernel(x, starts, pattern_i, pattern_q):
    raise NotImplementedError("write your pallas kernel here")


# speedup vs baseline: 10.3154x; 10.3154x over previous
import functools

import jax
import jax.numpy as jnp
from jax import lax
from jax.experimental import pallas as pl
from jax.experimental.pallas import tpu as pltpu
import jax.experimental.pallas.tpu_sc as plsc
from jax._src.pallas import mpmd as _mpmd

_B, _C, _T = 1024, 2, 16384
_SEG = 256
_KS = 9
_AMP = 0.08
_BB = 8
_GRID = _B // _BB
_NC, _NS = 2, 16
_NW = _NC * _NS
_SPW = _B // _NW
_WIN = 384


def _tc_body(x_ref, pi_ref, pq_ref, y_ref, amp_ref, pat_ref):
    xb = x_ref[...]
    y_ref[...] = xb
    ss = jnp.sum(xb * xb, axis=(1, 2))
    amp_ref[0, 0, :] = jnp.sqrt(ss / (_C * _T) + 1e-12)

    @pl.when(pl.program_id(0) == 0)
    def _():
        p = jnp.concatenate([pi_ref[...], pq_ref[...]], axis=0)
        pad = jnp.zeros((2, _KS // 2), dtype=p.dtype)
        pp = jnp.concatenate([pad, p, pad], axis=1)
        sm = pp[:, 0:_SEG]
        for k in range(1, _KS):
            sm = sm + pp[:, k:k + _SEG]
        sm = sm * (1.0 / _KS)
        sm = sm - jnp.mean(sm, axis=1, keepdims=True)
        rms = jnp.sqrt(jnp.mean(sm * sm) + 1e-8)
        pat_ref[...] = sm * (_AMP / rms)


_tc_call = pl.pallas_call(
    _tc_body,
    grid=(_GRID,),
    in_specs=[
        pl.BlockSpec((_BB, _C, _T), lambda i: (i, 0, 0)),
        pl.BlockSpec((1, _SEG), lambda i: (0, 0)),
        pl.BlockSpec((1, _SEG), lambda i: (0, 0)),
    ],
    out_specs=[
        pl.BlockSpec((_BB, _C, _T), lambda i: (i, 0, 0)),
        pl.BlockSpec((1, 1, _BB), lambda i: (i, 0, 0)),
        pl.BlockSpec((2, _SEG), lambda i: (0, 0)),
    ],
    out_shape=[
        jax.ShapeDtypeStruct((_B, _C, _T), jnp.float32),
        jax.ShapeDtypeStruct((_GRID, 1, _BB), jnp.float32),
        jax.ShapeDtypeStruct((2, _SEG), jnp.float32),
    ],
    compiler_params=pltpu.CompilerParams(
        dimension_semantics=("arbitrary",),
    ),
)


def _sc_body(y_in, amp_h, starts_h, pat_h, y_out,
             starts_v, amp_v, pat_v, win_v):
    del y_in
    cid = lax.axis_index("c")
    sid = lax.axis_index("s")
    wid = sid * _NC + cid
    b0 = wid * _SPW
    pltpu.sync_copy(starts_h.at[pl.ds(b0, _SPW)], starts_v)
    pltpu.sync_copy(amp_h.at[pl.ds(b0, _SPW)], amp_v)
    pltpu.sync_copy(pat_h, pat_v)
    iota = lax.iota(jnp.int32, 16)

    def sample(i, carry):
        lane_i = jnp.full((16,), i, jnp.int32)
        st = jnp.max(plsc.load_gather(starts_v, [lane_i]))
        av = plsc.load_gather(amp_v, [lane_i])
        b = b0 + i
        base = pl.multiple_of(
            jnp.minimum(jnp.bitwise_and(st, -128), _T - _WIN), 128)
        r = st - base
        pltpu.sync_copy(y_out.at[b, :, pl.ds(base, _WIN)], win_v)
        for ch in range(_C):
            idx0 = jnp.full((16,), ch, jnp.int32)
            for j in range(_SEG // 16):
                chunk = pat_v[pl.ds(ch * _SEG + j * 16, 16)]
                idx1 = iota + (r + j * 16)
                plsc.addupdate_scatter(win_v, [idx0, idx1], av * chunk)
        pltpu.sync_copy(win_v, y_out.at[b, :, pl.ds(base, _WIN)])
        return carry

    lax.fori_loop(0, _SPW, sample, 0)


_sc_mesh = plsc.VectorSubcoreMesh(
    core_axis_name="c", subcore_axis_name="s",
    num_cores=_NC, num_subcores=_NS,
)

_sc_call = _mpmd._mpmd_map(
    [(_sc_mesh, _sc_body)],
    out_types=jax.ShapeDtypeStruct((_B, _C, _T), jnp.float32),
    input_output_aliases={0: 0},
    compiler_params=pltpu.CompilerParams(needs_layout_passes=False),
    scratch_types=[
        pltpu.VMEM((_SPW,), jnp.int32),
        pltpu.VMEM((_SPW,), jnp.float32),
        pltpu.VMEM((2 * _SEG,), jnp.float32),
        pltpu.VMEM((_C, _WIN), jnp.float32),
    ],
)


@jax.jit
def kernel(x, starts, pattern_i, pattern_q):
    y, amp3, pat = _tc_call(
        x, pattern_i.reshape(1, _SEG), pattern_q.reshape(1, _SEG))
    out = _sc_call(
        y, amp3.reshape(_B), starts.astype(jnp.int32), pat.reshape(2 * _SEG))
    return out


# speedup vs baseline: 11.6749x; 1.1318x over previous
import functools

import jax
import jax.numpy as jnp
from jax import lax
from jax.experimental import pallas as pl
from jax.experimental.pallas import tpu as pltpu
import jax.experimental.pallas.tpu_sc as plsc
from jax._src.pallas import mpmd as _mpmd

_B, _C, _T = 1024, 2, 16384
_SEG = 256
_KS = 9
_AMP = 0.08
_BB = 8
_GRID = _B // _BB
_NC, _NS = 2, 16
_NW = _NC * _NS
_SPW = _B // _NW
_WIN = 384


def _tc_body(x_ref, pi_ref, pq_ref, y_ref, amp_ref, pat_ref):
    xb = x_ref[...]
    y_ref[...] = xb
    acc = xb * xb
    w = _T
    while w > 128:
        acc = acc[:, :, :w // 2] + acc[:, :, w // 2:w]
        w //= 2
    ss = jnp.sum(acc, axis=(1, 2))
    amp_ref[0, 0, :] = jnp.sqrt(ss / (_C * _T) + 1e-12)

    @pl.when(pl.program_id(0) == 0)
    def _():
        p = jnp.concatenate([pi_ref[...], pq_ref[...]], axis=0)
        pad = jnp.zeros((2, _KS // 2), dtype=p.dtype)
        pp = jnp.concatenate([pad, p, pad], axis=1)
        sm = pp[:, 0:_SEG]
        for k in range(1, _KS):
            sm = sm + pp[:, k:k + _SEG]
        sm = sm * (1.0 / _KS)
        sm = sm - jnp.mean(sm, axis=1, keepdims=True)
        rms = jnp.sqrt(jnp.mean(sm * sm) + 1e-8)
        pat_ref[...] = sm * (_AMP / rms)


_tc_call = pl.pallas_call(
    _tc_body,
    grid=(_GRID,),
    in_specs=[
        pl.BlockSpec((_BB, _C, _T), lambda i: (i, 0, 0)),
        pl.BlockSpec((1, _SEG), lambda i: (0, 0)),
        pl.BlockSpec((1, _SEG), lambda i: (0, 0)),
    ],
    out_specs=[
        pl.BlockSpec((_BB, _C, _T), lambda i: (i, 0, 0)),
        pl.BlockSpec((1, 1, _BB), lambda i: (i, 0, 0)),
        pl.BlockSpec((2, _SEG), lambda i: (0, 0)),
    ],
    out_shape=[
        jax.ShapeDtypeStruct((_B, _C, _T), jnp.float32),
        jax.ShapeDtypeStruct((_GRID, 1, _BB), jnp.float32),
        jax.ShapeDtypeStruct((2, _SEG), jnp.float32),
    ],
    compiler_params=pltpu.CompilerParams(
        dimension_semantics=("arbitrary",),
    ),
)


def _sc_body(y_in, amp_h, starts_h, pat_h, y_out,
             starts_v, amp_v, pat_v, win_v):
    del y_in
    cid = lax.axis_index("c")
    sid = lax.axis_index("s")
    wid = sid * _NC + cid
    b0 = wid * _SPW
    pltpu.sync_copy(starts_h.at[pl.ds(b0, _SPW)], starts_v)
    pltpu.sync_copy(amp_h.at[pl.ds(b0, _SPW)], amp_v)
    pltpu.sync_copy(pat_h, pat_v)
    iota = lax.iota(jnp.int32, 16)

    def sample(i, carry):
        lane_i = jnp.full((16,), i, jnp.int32)
        st = jnp.max(plsc.load_gather(starts_v, [lane_i]))
        av = plsc.load_gather(amp_v, [lane_i])
        b = b0 + i
        base = pl.multiple_of(
            jnp.minimum(jnp.bitwise_and(st, -128), _T - _WIN), 128)
        r = st - base
        pltpu.sync_copy(y_out.at[b, :, pl.ds(base, _WIN)], win_v)
        for ch in range(_C):
            idx0 = jnp.full((16,), ch, jnp.int32)
            for j in range(_SEG // 16):
                chunk = pat_v[pl.ds(ch * _SEG + j * 16, 16)]
                idx1 = iota + (r + j * 16)
                plsc.addupdate_scatter(win_v, [idx0, idx1], av * chunk)
        pltpu.sync_copy(win_v, y_out.at[b, :, pl.ds(base, _WIN)])
        return carry

    lax.fori_loop(0, _SPW, sample, 0)


_sc_mesh = plsc.VectorSubcoreMesh(
    core_axis_name="c", subcore_axis_name="s",
    num_cores=_NC, num_subcores=_NS,
)

_sc_call = _mpmd._mpmd_map(
    [(_sc_mesh, _sc_body)],
    out_types=jax.ShapeDtypeStruct((_B, _C, _T), jnp.float32),
    input_output_aliases={0: 0},
    compiler_params=pltpu.CompilerParams(needs_layout_passes=False),
    scratch_types=[
        pltpu.VMEM((_SPW,), jnp.int32),
        pltpu.VMEM((_SPW,), jnp.float32),
        pltpu.VMEM((2 * _SEG,), jnp.float32),
        pltpu.VMEM((_C, _WIN), jnp.float32),
    ],
)


@jax.jit
def kernel(x, starts, pattern_i, pattern_q):
    y, amp3, pat = _tc_call(
        x, pattern_i.reshape(1, _SEG), pattern_q.reshape(1, _SEG))
    out = _sc_call(
        y, amp3.reshape(_B), starts.astype(jnp.int32), pat.reshape(2 * _SEG))
    return out


# speedup vs baseline: 14.6478x; 1.2546x over previous
import functools

import jax
import jax.numpy as jnp
from jax import lax
from jax.experimental import pallas as pl
from jax.experimental.pallas import tpu as pltpu
import jax.experimental.pallas.tpu_sc as plsc
from jax._src.pallas import mpmd as _mpmd

_B, _C, _T = 1024, 2, 16384
_SEG = 256
_KS = 9
_AMP = 0.08
_BB = 16
_GRID = _B // _BB
_NC, _NS = 2, 16
_NW = _NC * _NS
_SPW = _B // _NW
_WIN = 384


def _tc_body(x_ref, pi_ref, pq_ref, y_ref, amp_ref, pat_ref):
    xb = x_ref[...]
    y_ref[...] = xb
    acc = xb * xb
    w = _T
    while w > 128:
        acc = acc[:, :, :w // 2] + acc[:, :, w // 2:w]
        w //= 2
    ss = jnp.sum(acc, axis=(1, 2))
    amp_ref[0, 0, :] = jnp.sqrt(ss / (_C * _T) + 1e-12)

    @pl.when(pl.program_id(0) == 0)
    def _():
        p = jnp.concatenate([pi_ref[...], pq_ref[...]], axis=0)
        pad = jnp.zeros((2, _KS // 2), dtype=p.dtype)
        pp = jnp.concatenate([pad, p, pad], axis=1)
        sm = pp[:, 0:_SEG]
        for k in range(1, _KS):
            sm = sm + pp[:, k:k + _SEG]
        sm = sm * (1.0 / _KS)
        sm = sm - jnp.mean(sm, axis=1, keepdims=True)
        rms = jnp.sqrt(jnp.mean(sm * sm) + 1e-8)
        pat_ref[...] = sm * (_AMP / rms)


_tc_call = pl.pallas_call(
    _tc_body,
    grid=(_GRID,),
    in_specs=[
        pl.BlockSpec((_BB, _C, _T), lambda i: (i, 0, 0)),
        pl.BlockSpec((1, _SEG), lambda i: (0, 0)),
        pl.BlockSpec((1, _SEG), lambda i: (0, 0)),
    ],
    out_specs=[
        pl.BlockSpec((_BB, _C, _T), lambda i: (i, 0, 0)),
        pl.BlockSpec((1, 1, _BB), lambda i: (i, 0, 0)),
        pl.BlockSpec((2, _SEG), lambda i: (0, 0)),
    ],
    out_shape=[
        jax.ShapeDtypeStruct((_B, _C, _T), jnp.float32),
        jax.ShapeDtypeStruct((_GRID, 1, _BB), jnp.float32),
        jax.ShapeDtypeStruct((2, _SEG), jnp.float32),
    ],
    compiler_params=pltpu.CompilerParams(
        dimension_semantics=("arbitrary",),
    ),
)


def _sc_body(y_in, amp_h, starts_h, pat_h, y_out,
             starts_v, amp_v, pat_v, win_v):
    del y_in
    cid = lax.axis_index("c")
    sid = lax.axis_index("s")
    wid = sid * _NC + cid
    b0 = wid * _SPW
    pltpu.sync_copy(starts_h.at[pl.ds(b0, _SPW)], starts_v)
    pltpu.sync_copy(amp_h.at[pl.ds(b0, _SPW)], amp_v)
    pltpu.sync_copy(pat_h, pat_v)
    iota = lax.iota(jnp.int32, 16)

    def sample(i, carry):
        lane_i = jnp.full((16,), i, jnp.int32)
        st = jnp.max(plsc.load_gather(starts_v, [lane_i]))
        av = plsc.load_gather(amp_v, [lane_i])
        b = b0 + i
        base = pl.multiple_of(
            jnp.minimum(jnp.bitwise_and(st, -128), _T - _WIN), 128)
        r = st - base
        pltpu.sync_copy(y_out.at[b, :, pl.ds(base, _WIN)], win_v)
        for ch in range(_C):
            idx0 = jnp.full((16,), ch, jnp.int32)
            for j in range(_SEG // 16):
                chunk = pat_v[pl.ds(ch * _SEG + j * 16, 16)]
                idx1 = iota + (r + j * 16)
                plsc.addupdate_scatter(win_v, [idx0, idx1], av * chunk)
        pltpu.sync_copy(win_v, y_out.at[b, :, pl.ds(base, _WIN)])
        return carry

    lax.fori_loop(0, _SPW, sample, 0)


_sc_mesh = plsc.VectorSubcoreMesh(
    core_axis_name="c", subcore_axis_name="s",
    num_cores=_NC, num_subcores=_NS,
)

_sc_call = _mpmd._mpmd_map(
    [(_sc_mesh, _sc_body)],
    out_types=jax.ShapeDtypeStruct((_B, _C, _T), jnp.float32),
    input_output_aliases={0: 0},
    compiler_params=pltpu.CompilerParams(needs_layout_passes=False),
    scratch_types=[
        pltpu.VMEM((_SPW,), jnp.int32),
        pltpu.VMEM((_SPW,), jnp.float32),
        pltpu.VMEM((2 * _SEG,), jnp.float32),
        pltpu.VMEM((_C, _WIN), jnp.float32),
    ],
)


@jax.jit
def kernel(x, starts, pattern_i, pattern_q):
    y, amp3, pat = _tc_call(
        x, pattern_i.reshape(1, _SEG), pattern_q.reshape(1, _SEG))
    out = _sc_call(
        y, amp3.reshape(_B), starts.astype(jnp.int32), pat.reshape(2 * _SEG))
    return out


# speedup vs baseline: 15.8646x; 1.0831x over previous
import functools

import jax
import jax.numpy as jnp
from jax import lax
from jax.experimental import pallas as pl
from jax.experimental.pallas import tpu as pltpu
import jax.experimental.pallas.tpu_sc as plsc
from jax._src.pallas import mpmd as _mpmd

_B, _C, _T = 1024, 2, 16384
_SEG = 256
_KS = 9
_AMP = 0.08
_BB = 32
_GRID = _B // _BB
_NC, _NS = 2, 16
_NW = _NC * _NS
_SPW = _B // _NW
_WIN = 384


def _tc_body(x_ref, pi_ref, pq_ref, y_ref, amp_ref, pat_ref):
    xb = x_ref[...]
    y_ref[...] = xb
    acc = xb * xb
    w = _T
    while w > 128:
        acc = acc[:, :, :w // 2] + acc[:, :, w // 2:w]
        w //= 2
    ss = jnp.sum(acc, axis=(1, 2))
    amp_ref[0, 0, :] = jnp.sqrt(ss / (_C * _T) + 1e-12)

    @pl.when(pl.program_id(0) == 0)
    def _():
        p = jnp.concatenate([pi_ref[...], pq_ref[...]], axis=0)
        pad = jnp.zeros((2, _KS // 2), dtype=p.dtype)
        pp = jnp.concatenate([pad, p, pad], axis=1)
        sm = pp[:, 0:_SEG]
        for k in range(1, _KS):
            sm = sm + pp[:, k:k + _SEG]
        sm = sm * (1.0 / _KS)
        sm = sm - jnp.mean(sm, axis=1, keepdims=True)
        rms = jnp.sqrt(jnp.mean(sm * sm) + 1e-8)
        pat_ref[...] = sm * (_AMP / rms)


_tc_call = pl.pallas_call(
    _tc_body,
    grid=(_GRID,),
    in_specs=[
        pl.BlockSpec((_BB, _C, _T), lambda i: (i, 0, 0)),
        pl.BlockSpec((1, _SEG), lambda i: (0, 0)),
        pl.BlockSpec((1, _SEG), lambda i: (0, 0)),
    ],
    out_specs=[
        pl.BlockSpec((_BB, _C, _T), lambda i: (i, 0, 0)),
        pl.BlockSpec((1, 1, _BB), lambda i: (i, 0, 0)),
        pl.BlockSpec((2, _SEG), lambda i: (0, 0)),
    ],
    out_shape=[
        jax.ShapeDtypeStruct((_B, _C, _T), jnp.float32),
        jax.ShapeDtypeStruct((_GRID, 1, _BB), jnp.float32),
        jax.ShapeDtypeStruct((2, _SEG), jnp.float32),
    ],
    compiler_params=pltpu.CompilerParams(
        dimension_semantics=("arbitrary",),
    ),
)


def _sc_body(y_in, amp_h, starts_h, pat_h, y_out,
             starts_v, amp_v, pat_v, win_v):
    del y_in
    cid = lax.axis_index("c")
    sid = lax.axis_index("s")
    wid = sid * _NC + cid
    b0 = wid * _SPW
    pltpu.sync_copy(starts_h.at[pl.ds(b0, _SPW)], starts_v)
    pltpu.sync_copy(amp_h.at[pl.ds(b0, _SPW)], amp_v)
    pltpu.sync_copy(pat_h, pat_v)
    iota = lax.iota(jnp.int32, 16)

    def sample(i, carry):
        lane_i = jnp.full((16,), i, jnp.int32)
        st = jnp.max(plsc.load_gather(starts_v, [lane_i]))
        av = plsc.load_gather(amp_v, [lane_i])
        b = b0 + i
        base = pl.multiple_of(
            jnp.minimum(jnp.bitwise_and(st, -128), _T - _WIN), 128)
        r = st - base
        pltpu.sync_copy(y_out.at[b, :, pl.ds(base, _WIN)], win_v)
        for ch in range(_C):
            idx0 = jnp.full((16,), ch, jnp.int32)
            for j in range(_SEG // 16):
                chunk = pat_v[pl.ds(ch * _SEG + j * 16, 16)]
                idx1 = iota + (r + j * 16)
                plsc.addupdate_scatter(win_v, [idx0, idx1], av * chunk)
        pltpu.sync_copy(win_v, y_out.at[b, :, pl.ds(base, _WIN)])
        return carry

    lax.fori_loop(0, _SPW, sample, 0)


_sc_mesh = plsc.VectorSubcoreMesh(
    core_axis_name="c", subcore_axis_name="s",
    num_cores=_NC, num_subcores=_NS,
)

_sc_call = _mpmd._mpmd_map(
    [(_sc_mesh, _sc_body)],
    out_types=jax.ShapeDtypeStruct((_B, _C, _T), jnp.float32),
    input_output_aliases={0: 0},
    compiler_params=pltpu.CompilerParams(needs_layout_passes=False),
    scratch_types=[
        pltpu.VMEM((_SPW,), jnp.int32),
        pltpu.VMEM((_SPW,), jnp.float32),
        pltpu.VMEM((2 * _SEG,), jnp.float32),
        pltpu.VMEM((_C, _WIN), jnp.float32),
    ],
)


@jax.jit
def kernel(x, starts, pattern_i, pattern_q):
    y, amp3, pat = _tc_call(
        x, pattern_i.reshape(1, _SEG), pattern_q.reshape(1, _SEG))
    out = _sc_call(
        y, amp3.reshape(_B), starts.astype(jnp.int32), pat.reshape(2 * _SEG))
    return out


# speedup vs baseline: 16.0141x; 1.0094x over previous
import functools

import jax
import jax.numpy as jnp
from jax import lax
from jax.experimental import pallas as pl
from jax.experimental.pallas import tpu as pltpu
import jax.experimental.pallas.tpu_sc as plsc
from jax._src.pallas import mpmd as _mpmd

_B, _C, _T = 1024, 2, 16384
_SEG = 256
_KS = 9
_AMP = 0.08
_BB = 64
_GRID = _B // _BB
_NC, _NS = 2, 16
_NW = _NC * _NS
_SPW = _B // _NW
_WIN = 384


def _tc_body(x_ref, pi_ref, pq_ref, y_ref, amp_ref, pat_ref):
    xb = x_ref[...]
    y_ref[...] = xb
    acc = xb * xb
    w = _T
    while w > 128:
        acc = acc[:, :, :w // 2] + acc[:, :, w // 2:w]
        w //= 2
    ss = jnp.sum(acc, axis=(1, 2))
    amp_ref[0, 0, :] = jnp.sqrt(ss / (_C * _T) + 1e-12)

    @pl.when(pl.program_id(0) == 0)
    def _():
        p = jnp.concatenate([pi_ref[...], pq_ref[...]], axis=0)
        pad = jnp.zeros((2, _KS // 2), dtype=p.dtype)
        pp = jnp.concatenate([pad, p, pad], axis=1)
        sm = pp[:, 0:_SEG]
        for k in range(1, _KS):
            sm = sm + pp[:, k:k + _SEG]
        sm = sm * (1.0 / _KS)
        sm = sm - jnp.mean(sm, axis=1, keepdims=True)
        rms = jnp.sqrt(jnp.mean(sm * sm) + 1e-8)
        pat_ref[...] = sm * (_AMP / rms)


_tc_call = pl.pallas_call(
    _tc_body,
    grid=(_GRID,),
    in_specs=[
        pl.BlockSpec((_BB, _C, _T), lambda i: (i, 0, 0)),
        pl.BlockSpec((1, _SEG), lambda i: (0, 0)),
        pl.BlockSpec((1, _SEG), lambda i: (0, 0)),
    ],
    out_specs=[
        pl.BlockSpec((_BB, _C, _T), lambda i: (i, 0, 0)),
        pl.BlockSpec((1, 1, _BB), lambda i: (i, 0, 0)),
        pl.BlockSpec((2, _SEG), lambda i: (0, 0)),
    ],
    out_shape=[
        jax.ShapeDtypeStruct((_B, _C, _T), jnp.float32),
        jax.ShapeDtypeStruct((_GRID, 1, _BB), jnp.float32),
        jax.ShapeDtypeStruct((2, _SEG), jnp.float32),
    ],
    compiler_params=pltpu.CompilerParams(
        dimension_semantics=("arbitrary",),
    ),
)


def _sc_body(y_in, amp_h, starts_h, pat_h, y_out,
             starts_v, amp_v, pat_v, win_v):
    del y_in
    cid = lax.axis_index("c")
    sid = lax.axis_index("s")
    wid = sid * _NC + cid
    b0 = wid * _SPW
    pltpu.sync_copy(starts_h.at[pl.ds(b0, _SPW)], starts_v)
    pltpu.sync_copy(amp_h.at[pl.ds(b0, _SPW)], amp_v)
    pltpu.sync_copy(pat_h, pat_v)
    iota = lax.iota(jnp.int32, 16)

    def sample(i, carry):
        lane_i = jnp.full((16,), i, jnp.int32)
        st = jnp.max(plsc.load_gather(starts_v, [lane_i]))
        av = plsc.load_gather(amp_v, [lane_i])
        b = b0 + i
        base = pl.multiple_of(
            jnp.minimum(jnp.bitwise_and(st, -128), _T - _WIN), 128)
        r = st - base
        pltpu.sync_copy(y_out.at[b, :, pl.ds(base, _WIN)], win_v)
        for ch in range(_C):
            idx0 = jnp.full((16,), ch, jnp.int32)
            for j in range(_SEG // 16):
                chunk = pat_v[pl.ds(ch * _SEG + j * 16, 16)]
                idx1 = iota + (r + j * 16)
                plsc.addupdate_scatter(win_v, [idx0, idx1], av * chunk)
        pltpu.sync_copy(win_v, y_out.at[b, :, pl.ds(base, _WIN)])
        return carry

    lax.fori_loop(0, _SPW, sample, 0)


_sc_mesh = plsc.VectorSubcoreMesh(
    core_axis_name="c", subcore_axis_name="s",
    num_cores=_NC, num_subcores=_NS,
)

_sc_call = _mpmd._mpmd_map(
    [(_sc_mesh, _sc_body)],
    out_types=jax.ShapeDtypeStruct((_B, _C, _T), jnp.float32),
    input_output_aliases={0: 0},
    compiler_params=pltpu.CompilerParams(needs_layout_passes=False),
    scratch_types=[
        pltpu.VMEM((_SPW,), jnp.int32),
        pltpu.VMEM((_SPW,), jnp.float32),
        pltpu.VMEM((2 * _SEG,), jnp.float32),
        pltpu.VMEM((_C, _WIN), jnp.float32),
    ],
)


@jax.jit
def kernel(x, starts, pattern_i, pattern_q):
    y, amp3, pat = _tc_call(
        x, pattern_i.reshape(1, _SEG), pattern_q.reshape(1, _SEG))
    out = _sc_call(
        y, amp3.reshape(_B), starts.astype(jnp.int32), pat.reshape(2 * _SEG))
    return out


# speedup vs baseline: 18.5634x; 1.1592x over previous
import functools

import jax
import jax.numpy as jnp
from jax import lax
from jax.experimental import pallas as pl
from jax.experimental.pallas import tpu as pltpu
import jax.experimental.pallas.tpu_sc as plsc
from jax._src.pallas import mpmd as _mpmd

_B, _C, _T = 1024, 2, 16384
_SEG = 256
_KS = 9
_AMP = 0.08
_BB = 64
_GRID = _B // _BB
_NC, _NS = 2, 16
_NW = _NC * _NS
_SPW = _B // _NW
_WIN = 384


def _tc_body(x_ref, pi_ref, pq_ref, y_ref, amp_ref, pat_ref):
    xb = x_ref[...]
    y_ref[...] = xb
    acc = xb * xb
    w = _T
    while w > 128:
        acc = acc[:, :, :w // 2] + acc[:, :, w // 2:w]
        w //= 2
    ss = jnp.sum(acc, axis=(1, 2))
    amp_ref[0, 0, :] = jnp.sqrt(ss / (_C * _T) + 1e-12)

    @pl.when(pl.program_id(0) == 0)
    def _():
        p = jnp.concatenate([pi_ref[...], pq_ref[...]], axis=0)
        pad = jnp.zeros((2, _KS // 2), dtype=p.dtype)
        pp = jnp.concatenate([pad, p, pad], axis=1)
        sm = pp[:, 0:_SEG]
        for k in range(1, _KS):
            sm = sm + pp[:, k:k + _SEG]
        sm = sm * (1.0 / _KS)
        sm = sm - jnp.mean(sm, axis=1, keepdims=True)
        rms = jnp.sqrt(jnp.mean(sm * sm) + 1e-8)
        pat_ref[...] = sm * (_AMP / rms)


_tc_call = pl.pallas_call(
    _tc_body,
    grid=(_GRID,),
    in_specs=[
        pl.BlockSpec((_BB, _C, _T), lambda i: (i, 0, 0)),
        pl.BlockSpec((1, _SEG), lambda i: (0, 0)),
        pl.BlockSpec((1, _SEG), lambda i: (0, 0)),
    ],
    out_specs=[
        pl.BlockSpec((_BB, _C, _T), lambda i: (i, 0, 0)),
        pl.BlockSpec((1, 1, _BB), lambda i: (i, 0, 0)),
        pl.BlockSpec((2, _SEG), lambda i: (0, 0)),
    ],
    out_shape=[
        jax.ShapeDtypeStruct((_B, _C, _T), jnp.float32),
        jax.ShapeDtypeStruct((_GRID, 1, _BB), jnp.float32),
        jax.ShapeDtypeStruct((2, _SEG), jnp.float32),
    ],
    compiler_params=pltpu.CompilerParams(
        dimension_semantics=("arbitrary",),
    ),
)


def _sc_body(y_in, amp_h, starts_h, pat_h, y_out,
             starts_v, amp_v, pat_v, win_all, sem):
    del y_in
    cid = lax.axis_index("c")
    sid = lax.axis_index("s")
    wid = sid * _NC + cid
    b0 = wid * _SPW
    pltpu.sync_copy(starts_h.at[pl.ds(b0, _SPW)], starts_v)
    pltpu.sync_copy(amp_h.at[pl.ds(b0, _SPW)], amp_v)
    pltpu.sync_copy(pat_h, pat_v)
    iota = lax.iota(jnp.int32, 16)

    def start_of(i):
        lane_i = jnp.full((16,), i, jnp.int32)
        st = jnp.max(plsc.load_gather(starts_v, [lane_i]))
        base = pl.multiple_of(
            jnp.minimum(jnp.bitwise_and(st, -128), _T - _WIN), 128)
        return st, base

    def fire_in(i, carry):
        _, base = start_of(i)
        pltpu.async_copy(
            y_out.at[b0 + i, :, pl.ds(base, _WIN)], win_all.at[i], sem)
        return carry

    def drain(i, carry):
        pltpu.make_async_copy(
            y_out.at[b0, :, pl.ds(0, _WIN)], win_all.at[0], sem).wait()
        return carry

    def compute(i, carry):
        lane_i = jnp.full((16,), i, jnp.int32)
        st, base = start_of(i)
        r = st - base
        av = plsc.load_gather(amp_v, [lane_i])
        for ch in range(_C):
            idx0 = jnp.full((16,), ch, jnp.int32)
            for j in range(_SEG // 16):
                chunk = pat_v[pl.ds(ch * _SEG + j * 16, 16)]
                idx1 = iota + (r + j * 16)
                plsc.addupdate_scatter(win_all.at[i], [idx0, idx1], av * chunk)
        return carry

    def fire_out(i, carry):
        _, base = start_of(i)
        pltpu.async_copy(
            win_all.at[i], y_out.at[b0 + i, :, pl.ds(base, _WIN)], sem)
        return carry

    lax.fori_loop(0, _SPW, fire_in, 0)
    lax.fori_loop(0, _SPW, drain, 0)
    lax.fori_loop(0, _SPW, compute, 0)
    lax.fori_loop(0, _SPW, fire_out, 0)
    lax.fori_loop(0, _SPW, drain, 0)


_sc_mesh = plsc.VectorSubcoreMesh(
    core_axis_name="c", subcore_axis_name="s",
    num_cores=_NC, num_subcores=_NS,
)

_sc_call = _mpmd._mpmd_map(
    [(_sc_mesh, _sc_body)],
    out_types=jax.ShapeDtypeStruct((_B, _C, _T), jnp.float32),
    input_output_aliases={0: 0},
    compiler_params=pltpu.CompilerParams(needs_layout_passes=False),
    scratch_types=[
        pltpu.VMEM((_SPW,), jnp.int32),
        pltpu.VMEM((_SPW,), jnp.float32),
        pltpu.VMEM((2 * _SEG,), jnp.float32),
        pltpu.VMEM((_SPW, _C, _WIN), jnp.float32),
        pltpu.SemaphoreType.DMA,
    ],
)


@jax.jit
def kernel(x, starts, pattern_i, pattern_q):
    y, amp3, pat = _tc_call(
        x, pattern_i.reshape(1, _SEG), pattern_q.reshape(1, _SEG))
    out = _sc_call(
        y, amp3.reshape(_B), starts.astype(jnp.int32), pat.reshape(2 * _SEG))
    return out


# speedup vs baseline: 18.6089x; 1.0025x over previous
import functools

import jax
import jax.numpy as jnp
from jax import lax
from jax.experimental import pallas as pl
from jax.experimental.pallas import tpu as pltpu
import jax.experimental.pallas.tpu_sc as plsc
from jax._src.pallas import mpmd as _mpmd

_B, _C, _T = 1024, 2, 16384
_SEG = 256
_KS = 9
_AMP = 0.08
_BB = 64
_GRID = _B // _BB
_NC, _NS = 2, 16
_NW = _NC * _NS
_SPW = _B // _NW
_WIN = 384


def _tc_body(x_ref, pi_ref, pq_ref, y_ref, amp_ref, pat_ref):
    xb = x_ref[...]
    y_ref[...] = xb
    acc = xb * xb
    w = _T
    while w > 128:
        acc = acc[:, :, :w // 2] + acc[:, :, w // 2:w]
        w //= 2
    ss = jnp.sum(acc, axis=(1, 2))
    amp_ref[0, 0, :] = jnp.sqrt(ss / (_C * _T) + 1e-12)

    @pl.when(pl.program_id(0) == 0)
    def _():
        p = jnp.concatenate([pi_ref[...], pq_ref[...]], axis=0)
        pad = jnp.zeros((2, _KS // 2), dtype=p.dtype)
        pp = jnp.concatenate([pad, p, pad], axis=1)
        sm = pp[:, 0:_SEG]
        for k in range(1, _KS):
            sm = sm + pp[:, k:k + _SEG]
        sm = sm * (1.0 / _KS)
        sm = sm - jnp.mean(sm, axis=1, keepdims=True)
        rms = jnp.sqrt(jnp.mean(sm * sm) + 1e-8)
        pat_ref[...] = sm * (_AMP / rms)


_tc_call = pl.pallas_call(
    _tc_body,
    grid=(_GRID,),
    in_specs=[
        pl.BlockSpec((_BB, _C, _T), lambda i: (i, 0, 0)),
        pl.BlockSpec((1, _SEG), lambda i: (0, 0)),
        pl.BlockSpec((1, _SEG), lambda i: (0, 0)),
    ],
    out_specs=[
        pl.BlockSpec((_BB, _C, _T), lambda i: (i, 0, 0)),
        pl.BlockSpec((1, 1, _BB), lambda i: (i, 0, 0)),
        pl.BlockSpec((2, _SEG), lambda i: (0, 0)),
    ],
    out_shape=[
        jax.ShapeDtypeStruct((_B, _C, _T), jnp.float32),
        jax.ShapeDtypeStruct((_GRID, 1, _BB), jnp.float32),
        jax.ShapeDtypeStruct((2, _SEG), jnp.float32),
    ],
    compiler_params=pltpu.CompilerParams(
        dimension_semantics=("arbitrary",),
        vmem_limit_bytes=120 * 1024 * 1024,
    ),
)


def _sc_body(y_in, amp_h, starts_h, pat_h, y_out,
             starts_v, amp_v, pat_v, win_all, base_v, rel_v, sem):
    del y_in
    cid = lax.axis_index("c")
    sid = lax.axis_index("s")
    wid = sid * _NC + cid
    b0 = wid * _SPW
    pltpu.sync_copy(starts_h.at[pl.ds(b0, _SPW)], starts_v)
    pltpu.sync_copy(amp_h.at[pl.ds(b0, _SPW)], amp_v)
    pltpu.sync_copy(pat_h, pat_v)
    iota = lax.iota(jnp.int32, 16)
    for k in range(_SPW // 16):
        sv = starts_v[pl.ds(k * 16, 16)]
        bv = jnp.minimum(jnp.bitwise_and(sv, -128), _T - _WIN)
        base_v[pl.ds(k * 16, 16)] = bv
        rel_v[pl.ds(k * 16, 16)] = sv - bv

    def start_of(i):
        base = pl.multiple_of(base_v[pl.ds(i, 16)][0], 128)
        return base

    def fire_in(i, carry):
        base = start_of(i)
        pltpu.async_copy(
            y_out.at[b0 + i, :, pl.ds(base, _WIN)], win_all.at[i], sem)
        return carry

    def drain(i, carry):
        pltpu.make_async_copy(
            y_out.at[b0, :, pl.ds(0, _WIN)], win_all.at[0], sem).wait()
        return carry

    def compute(i, carry):
        lane_i = jnp.full((16,), i, jnp.int32)
        r = rel_v[pl.ds(i, 16)][0]
        av = plsc.load_gather(amp_v, [lane_i])
        for ch in range(_C):
            idx0 = jnp.full((16,), ch, jnp.int32)
            for j in range(_SEG // 16):
                chunk = pat_v[pl.ds(ch * _SEG + j * 16, 16)]
                idx1 = iota + (r + j * 16)
                plsc.addupdate_scatter(win_all.at[i], [idx0, idx1], av * chunk)
        return carry

    def fire_out(i, carry):
        base = start_of(i)
        pltpu.async_copy(
            win_all.at[i], y_out.at[b0 + i, :, pl.ds(base, _WIN)], sem)
        return carry

    lax.fori_loop(0, _SPW, fire_in, 0)
    lax.fori_loop(0, _SPW, drain, 0)
    lax.fori_loop(0, _SPW, compute, 0)
    lax.fori_loop(0, _SPW, fire_out, 0)
    lax.fori_loop(0, _SPW, drain, 0)


_sc_mesh = plsc.VectorSubcoreMesh(
    core_axis_name="c", subcore_axis_name="s",
    num_cores=_NC, num_subcores=_NS,
)

_sc_call = _mpmd._mpmd_map(
    [(_sc_mesh, _sc_body)],
    out_types=jax.ShapeDtypeStruct((_B, _C, _T), jnp.float32),
    input_output_aliases={0: 0},
    compiler_params=pltpu.CompilerParams(needs_layout_passes=False),
    scratch_types=[
        pltpu.VMEM((_SPW,), jnp.int32),
        pltpu.VMEM((_SPW,), jnp.float32),
        pltpu.VMEM((2 * _SEG,), jnp.float32),
        pltpu.VMEM((_SPW, _C, _WIN), jnp.float32),
        pltpu.VMEM((_SPW + 16,), jnp.int32),
        pltpu.VMEM((_SPW + 16,), jnp.int32),
        pltpu.SemaphoreType.DMA,
    ],
)


@jax.jit
def kernel(x, starts, pattern_i, pattern_q):
    y, amp3, pat = _tc_call(
        x, pattern_i.reshape(1, _SEG), pattern_q.reshape(1, _SEG))
    out = _sc_call(
        y, amp3.reshape(_B), starts.astype(jnp.int32), pat.reshape(2 * _SEG))
    return out


# speedup vs baseline: 18.6450x; 1.0019x over previous
import jax
import jax.numpy as jnp
from jax import lax
from jax.experimental import pallas as pl
from jax.experimental.pallas import tpu as pltpu
import jax.experimental.pallas.tpu_sc as plsc
from jax._src.pallas import mpmd as _mpmd

_B, _C, _T = 1024, 2, 16384
_SEG = 256
_KS = 9
_AMP = 0.08
_BB = 64
_GRID = _B // _BB
_NC, _NS = 2, 16
_NW = _NC * _NS
_SPW = _B // _NW
_WIN = 384


def _tc_body(x_ref, pi_ref, pq_ref, y_ref, amp_ref, pat_ref):
    xb = x_ref[...]
    y_ref[...] = xb
    acc = xb * xb
    w = _T
    while w > 128:
        acc = acc[:, :, :w // 2] + acc[:, :, w // 2:w]
        w //= 2
    ss = jnp.sum(acc, axis=(1, 2))
    amp_ref[0, 0, :] = jnp.sqrt(ss / (_C * _T) + 1e-12)

    @pl.when(pl.program_id(0) == 0)
    def _():
        p = jnp.concatenate([pi_ref[...], pq_ref[...]], axis=0)
        pad = jnp.zeros((2, _KS // 2), dtype=p.dtype)
        pp = jnp.concatenate([pad, p, pad], axis=1)
        sm = pp[:, 0:_SEG]
        for k in range(1, _KS):
            sm = sm + pp[:, k:k + _SEG]
        sm = sm * (1.0 / _KS)
        sm = sm - jnp.mean(sm, axis=1, keepdims=True)
        rms = jnp.sqrt(jnp.mean(sm * sm) + 1e-8)
        pat_ref[...] = sm * (_AMP / rms)


_tc_call = pl.pallas_call(
    _tc_body,
    grid=(_GRID,),
    in_specs=[
        pl.BlockSpec((_BB, _C, _T), lambda i: (i, 0, 0)),
        pl.BlockSpec((1, _SEG), lambda i: (0, 0)),
        pl.BlockSpec((1, _SEG), lambda i: (0, 0)),
    ],
    out_specs=[
        pl.BlockSpec((_BB, _C, _T), lambda i: (i, 0, 0)),
        pl.BlockSpec((1, 1, _BB), lambda i: (i, 0, 0)),
        pl.BlockSpec((2, _SEG), lambda i: (0, 0)),
    ],
    out_shape=[
        jax.ShapeDtypeStruct((_B, _C, _T), jnp.float32),
        jax.ShapeDtypeStruct((_GRID, 1, _BB), jnp.float32),
        jax.ShapeDtypeStruct((2, _SEG), jnp.float32),
    ],
    compiler_params=pltpu.CompilerParams(
        dimension_semantics=("arbitrary",),
        vmem_limit_bytes=120 * 1024 * 1024,
    ),
)


def _sc_body(y_in, amp_h, starts_h, pat_h, y_out,
             starts_v, amp_v, pat_v, win_all, base_v, rel_v, sem):
    del y_in
    cid = lax.axis_index("c")
    sid = lax.axis_index("s")
    wid = sid * _NC + cid
    b0 = wid * _SPW
    pltpu.sync_copy(starts_h.at[pl.ds(b0, _SPW)], starts_v)
    pltpu.sync_copy(amp_h.at[pl.ds(b0, _SPW)], amp_v)
    pltpu.sync_copy(pat_h, pat_v)
    iota = lax.iota(jnp.int32, 16)
    for k in range(_SPW // 16):
        sv = starts_v[pl.ds(k * 16, 16)]
        bv = jnp.minimum(jnp.bitwise_and(sv, -128), _T - _WIN)
        base_v[pl.ds(k * 16, 16)] = bv
        rel_v[pl.ds(k * 16, 16)] = sv - bv

    def start_of(i):
        base = pl.multiple_of(base_v[pl.ds(i, 16)][0], 128)
        return base

    def fire_in(i, carry):
        base = start_of(i)
        pltpu.async_copy(
            y_out.at[b0 + i, :, pl.ds(base, _WIN)], win_all.at[i], sem)
        return carry

    def drain(i, carry):
        pltpu.make_async_copy(
            y_out.at[b0, :, pl.ds(0, _WIN)], win_all.at[0], sem).wait()
        return carry

    def compute(i, carry):
        lane_i = jnp.full((16,), i, jnp.int32)
        r = rel_v[pl.ds(i, 16)][0]
        av = plsc.load_gather(amp_v, [lane_i])
        for ch in range(_C):
            idx0 = jnp.full((16,), ch, jnp.int32)
            for j in range(_SEG // 16):
                chunk = pat_v[pl.ds(ch * _SEG + j * 16, 16)]
                idx1 = iota + (r + j * 16)
                plsc.addupdate_scatter(win_all.at[i], [idx0, idx1], av * chunk)
        return carry

    def fire_out(i, carry):
        base = start_of(i)
        pltpu.async_copy(
            win_all.at[i], y_out.at[b0 + i, :, pl.ds(base, _WIN)], sem)
        return carry

    lax.fori_loop(0, _SPW, fire_in, 0)
    lax.fori_loop(0, _SPW, drain, 0)
    lax.fori_loop(0, _SPW, compute, 0)
    lax.fori_loop(0, _SPW, fire_out, 0)
    lax.fori_loop(0, _SPW, drain, 0)


_sc_mesh = plsc.VectorSubcoreMesh(
    core_axis_name="c", subcore_axis_name="s",
    num_cores=_NC, num_subcores=_NS,
)

_sc_call = _mpmd._mpmd_map(
    [(_sc_mesh, _sc_body)],
    out_types=jax.ShapeDtypeStruct((_B, _C, _T), jnp.float32),
    input_output_aliases={0: 0},
    compiler_params=pltpu.CompilerParams(needs_layout_passes=False),
    scratch_types=[
        pltpu.VMEM((_SPW,), jnp.int32),
        pltpu.VMEM((_SPW,), jnp.float32),
        pltpu.VMEM((2 * _SEG,), jnp.float32),
        pltpu.VMEM((_SPW, _C, _WIN), jnp.float32),
        pltpu.VMEM((_SPW + 16,), jnp.int32),
        pltpu.VMEM((_SPW + 16,), jnp.int32),
        pltpu.SemaphoreType.DMA,
    ],
)


@jax.jit
def kernel(x, starts, pattern_i, pattern_q):
    y, amp3, pat = _tc_call(
        x, pattern_i.reshape(1, _SEG), pattern_q.reshape(1, _SEG))
    out = _sc_call(
        y, amp3.reshape(_B), starts.astype(jnp.int32), pat.reshape(2 * _SEG))
    return out
